# Initial kernel scaffold; baseline (speedup 1.0000x reference)
import functools

import jax
import jax.numpy as jnp
from jax import lax
from jax.experimental import pallas as pl
from jax.experimental.pallas import tpu as pltpu
from jax.experimental.pallas import tpu_sc as plsc

_N = 10000
_E = 320000
_CH = 32000


def _match_body(ss_hbm, ds_hbm, match_hbm, match_v, ssv, dsv):
    c = lax.axis_index("c")
    s = lax.axis_index("s")

    @pl.when((c == 0) & (s == 0))
    def _():
        neg1 = jnp.full((16,), -1, jnp.int32)

        def init(i, carry):
            match_v[pl.ds(i * 16, 16)] = neg1
            return carry

        lax.fori_loop(0, _N // 16, init, 0)

        def chunk(ci, carry):
            pltpu.sync_copy(ss_hbm.at[pl.ds(ci * _CH, _CH)], ssv)
            pltpu.sync_copy(ds_hbm.at[pl.ds(ci * _CH, _CH)], dsv)

            def edge(e, c2):
                u = ssv[e]
                v = dsv[e]
                mu = match_v[u]
                mv = match_v[v]
                take = (mu == -1) & (v > u) & (mv == -1)
                match_v[u] = jnp.where(take, u, mu)
                match_v[v] = jnp.where(take, u, mv)
                return c2

            lax.fori_loop(0, _CH, edge, 0)
            return carry

        lax.fori_loop(0, _E // _CH, chunk, 0)
        pltpu.sync_copy(match_v, match_hbm)


@jax.jit
def _sc_match(ss, ds):
    mesh = plsc.VectorSubcoreMesh(core_axis_name="c", subcore_axis_name="s")
    return pl.kernel(
        _match_body,
        mesh=mesh,
        out_type=jax.ShapeDtypeStruct((_N,), jnp.int32),
        scratch_types=[
            pltpu.VMEM((_N,), jnp.int32),
            pltpu.VMEM((_CH,), jnp.int32),
            pltpu.VMEM((_CH,), jnp.int32),
        ],
    )(ss, ds)


def kernel(x, edge_index, batch, W1, b1, W2, b2, fc1_w, fc1_b, fc2_w, fc2_b):
    n = x.shape[0]
    idx = jnp.arange(n, dtype=jnp.int32)
    src = edge_index[0]
    dst = edge_index[1]
    order = jnp.argsort(src)
    ss = src[order]
    ds = dst[order]

    match1 = _sc_match(ss, ds)
    cluster1 = jnp.where(match1 == -1, idx, match1)
    head1 = cluster1 == idx
    h1i = head1.astype(jnp.int32)
    rank1 = jnp.cumsum(h1i) - h1i
    n1 = jnp.sum(h1i)
    inv1 = rank1[cluster1]
    partner1 = idx.at[jnp.where(head1, n, cluster1)].set(idx, mode="drop")

    e0 = inv1[ss]
    e1 = inv1[ds]
    valid = e0 != e1
    big = jnp.int32(n * n)
    k = jnp.where(valid, e0 * n1 + e1, big)
    ks = jnp.sort(k)
    kv = ks < big
    first = jnp.concatenate([jnp.ones((1,), bool), ks[1:] != ks[:-1]])
    keep = first & kv
    es = jnp.where(kv, ks // n1, 0).astype(jnp.int32)
    ed = jnp.where(kv, ks % n1, 0).astype(jnp.int32)

    match2 = _sc_match(es, ed)
    cluster2 = jnp.where(match2 == -1, idx, match2)
    head2 = (cluster2 == idx) & (idx < n1)
    n2 = jnp.sum(head2.astype(jnp.int32))
    partner2 = idx.at[jnp.where(head2 | (idx >= n1), n, cluster2)].set(
        idx, mode="drop")

    xf = x.astype(jnp.float32)
    deg1 = jnp.zeros((n,), jnp.float32).at[ss].add(1.0)
    dis1 = jnp.where(deg1 > 0, lax.rsqrt(jnp.maximum(deg1, 1e-12)), 0.0)
    coef1 = -dis1[ss] * dis1[ds]
    y = xf @ W1[1]
    t1 = jnp.zeros_like(y).at[ds].add(coef1[:, None] * y[ss])
    h = jax.nn.relu(xf @ W1[0] + t1 + b1)

    hp_nodes = jnp.maximum(h, h[partner1])
    hp = jnp.zeros_like(h).at[jnp.where(head1, rank1, n)].set(
        hp_nodes, mode="drop")

    w2v = keep.astype(jnp.float32)
    deg2 = jnp.zeros((n,), jnp.float32).at[es].add(w2v)
    dis2 = jnp.where(deg2 > 0, lax.rsqrt(jnp.maximum(deg2, 1e-12)), 0.0)
    coef2 = -dis2[es] * dis2[ed] * w2v
    t2 = jnp.zeros_like(hp).at[ed].add(coef2[:, None] * hp[es])
    h2 = jax.nn.relu(hp @ W2[0] + t2 @ W2[1] + b2)

    h2p = jnp.maximum(h2, h2[partner2])
    g = jnp.sum(jnp.where(head2[:, None], h2p, 0.0), axis=0,
                keepdims=True) / n2.astype(jnp.float32)

    g = jax.nn.relu(g @ fc1_w + fc1_b)
    return g @ fc2_w + fc2_b


# speedup vs baseline: 709.4486x; 709.4486x over previous
import functools

import jax
import jax.numpy as jnp
from jax import lax
from jax.experimental import pallas as pl
from jax.experimental.pallas import tpu as pltpu
from jax.experimental.pallas import tpu_sc as plsc

_N = 10000
_E = 320000
_CH = 32000


def _match_body(ss_hbm, ds_hbm, match_hbm, match_v, ssv, dsv):
    c = lax.axis_index("c")
    s = lax.axis_index("s")

    @pl.when((c == 0) & (s == 0))
    def _():
        lane = lax.iota(jnp.int32, 16)
        neg1 = jnp.full((16,), -1, jnp.int32)
        sixteen = jnp.full((16,), 16, jnp.int32)

        def init(i, carry):
            match_v[pl.ds(i * 16, 16)] = neg1
            return carry

        lax.fori_loop(0, _N // 16, init, 0)

        def chunk(ci, carry):
            pltpu.sync_copy(ss_hbm.at[pl.ds(ci * _CH, _CH)], ssv)
            pltpu.sync_copy(ds_hbm.at[pl.ds(ci * _CH, _CH)], dsv)

            def vec(i, c2):
                uvec = ssv[pl.ds(i * 16, 16)]
                vvec = dsv[pl.ds(i * 16, 16)]
                gt = vvec > uvec

                def wbody(minlane):
                    mu = plsc.load_gather(match_v, [uvec])
                    mv = plsc.load_gather(match_v, [vvec])
                    cand = ((mu == neg1) & gt & (mv == neg1)
                            & (lane >= minlane))
                    has = jnp.any(cand)
                    j0 = plsc.all_reduce_ffs(cand) + jnp.zeros(
                        (16,), jnp.int32)
                    m0 = cand & (lane == j0)
                    plsc.store_scatter(match_v, [uvec], uvec, mask=m0)
                    plsc.store_scatter(match_v, [vvec], uvec, mask=m0)
                    return jnp.where(has, j0[0] + 1, jnp.int32(16))

                lax.while_loop(lambda ml: ml < 16, wbody, jnp.int32(0))
                return c2

            lax.fori_loop(0, _CH // 16, vec, 0)
            return carry

        lax.fori_loop(0, _E // _CH, chunk, 0)
        pltpu.sync_copy(match_v, match_hbm)


@jax.jit
def _sc_match(ss, ds):
    mesh = plsc.VectorSubcoreMesh(core_axis_name="c", subcore_axis_name="s")
    return pl.kernel(
        _match_body,
        mesh=mesh,
        compiler_params=pltpu.CompilerParams(needs_layout_passes=False),
        out_type=jax.ShapeDtypeStruct((_N,), jnp.int32),
        scratch_types=[
            pltpu.VMEM((_N,), jnp.int32),
            pltpu.VMEM((_CH,), jnp.int32),
            pltpu.VMEM((_CH,), jnp.int32),
        ],
    )(ss, ds)


def kernel(x, edge_index, batch, W1, b1, W2, b2, fc1_w, fc1_b, fc2_w, fc2_b):
    n = x.shape[0]
    idx = jnp.arange(n, dtype=jnp.int32)
    src = edge_index[0]
    dst = edge_index[1]
    order = jnp.argsort(src)
    ss = src[order]
    ds = dst[order]

    match1 = _sc_match(ss, ds)
    cluster1 = jnp.where(match1 == -1, idx, match1)
    head1 = cluster1 == idx
    h1i = head1.astype(jnp.int32)
    rank1 = jnp.cumsum(h1i) - h1i
    n1 = jnp.sum(h1i)
    inv1 = rank1[cluster1]
    partner1 = idx.at[jnp.where(head1, n, cluster1)].set(idx, mode="drop")

    e0 = inv1[ss]
    e1 = inv1[ds]
    valid = e0 != e1
    big = jnp.int32(n * n)
    k = jnp.where(valid, e0 * n1 + e1, big)
    ks = jnp.sort(k)
    kv = ks < big
    first = jnp.concatenate([jnp.ones((1,), bool), ks[1:] != ks[:-1]])
    keep = first & kv
    es = jnp.where(kv, ks // n1, 0).astype(jnp.int32)
    ed = jnp.where(kv, ks % n1, 0).astype(jnp.int32)

    match2 = _sc_match(es, ed)
    cluster2 = jnp.where(match2 == -1, idx, match2)
    head2 = (cluster2 == idx) & (idx < n1)
    n2 = jnp.sum(head2.astype(jnp.int32))
    partner2 = idx.at[jnp.where(head2 | (idx >= n1), n, cluster2)].set(
        idx, mode="drop")

    xf = x.astype(jnp.float32)
    deg1 = jnp.zeros((n,), jnp.float32).at[ss].add(1.0)
    dis1 = jnp.where(deg1 > 0, lax.rsqrt(jnp.maximum(deg1, 1e-12)), 0.0)
    coef1 = -dis1[ss] * dis1[ds]
    y = xf @ W1[1]
    t1 = jnp.zeros_like(y).at[ds].add(coef1[:, None] * y[ss])
    h = jax.nn.relu(xf @ W1[0] + t1 + b1)

    hp_nodes = jnp.maximum(h, h[partner1])
    hp = jnp.zeros_like(h).at[jnp.where(head1, rank1, n)].set(
        hp_nodes, mode="drop")

    w2v = keep.astype(jnp.float32)
    deg2 = jnp.zeros((n,), jnp.float32).at[es].add(w2v)
    dis2 = jnp.where(deg2 > 0, lax.rsqrt(jnp.maximum(deg2, 1e-12)), 0.0)
    coef2 = -dis2[es] * dis2[ed] * w2v
    t2 = jnp.zeros_like(hp).at[ed].add(coef2[:, None] * hp[es])
    h2 = jax.nn.relu(hp @ W2[0] + t2 @ W2[1] + b2)

    h2p = jnp.maximum(h2, h2[partner2])
    g = jnp.sum(jnp.where(head2[:, None], h2p, 0.0), axis=0,
                keepdims=True) / n2.astype(jnp.float32)

    g = jax.nn.relu(g @ fc1_w + fc1_b)
    return g @ fc2_w + fc2_b


# speedup vs baseline: 2407.1315x; 3.3930x over previous
import jax
import jax.numpy as jnp
from jax import lax
from jax.experimental import pallas as pl
from jax.experimental.pallas import tpu as pltpu
from jax.experimental.pallas import tpu_sc as plsc

_N = 10000
_E = 320000
_CH = 32000
_D = 32
_NW = 32
_EPW = _E // _NW
_C = 1000


def _match_body(ss_hbm, ds_hbm, match_hbm, match_v, ssv, dsv):
    c = lax.axis_index("c")
    s = lax.axis_index("s")

    @pl.when((c == 0) & (s == 0))
    def _():
        lane = lax.iota(jnp.int32, 16)
        neg1 = jnp.full((16,), -1, jnp.int32)

        def init(i, carry):
            match_v[pl.ds(i * 16, 16)] = neg1
            return carry

        lax.fori_loop(0, _N // 16, init, 0)

        def chunk(ci, carry):
            pltpu.sync_copy(ss_hbm.at[pl.ds(ci * _CH, _CH)], ssv)
            pltpu.sync_copy(ds_hbm.at[pl.ds(ci * _CH, _CH)], dsv)

            def vec(i, c2):
                uvec = ssv[pl.ds(i * 16, 16)]
                vvec = dsv[pl.ds(i * 16, 16)]
                gt = vvec > uvec

                def wbody(minlane):
                    mu = plsc.load_gather(match_v, [uvec])
                    mv = plsc.load_gather(match_v, [vvec])
                    cand = ((mu == neg1) & gt & (mv == neg1)
                            & (lane >= minlane))
                    has = jnp.any(cand)
                    j0 = plsc.all_reduce_ffs(cand) + jnp.zeros(
                        (16,), jnp.int32)
                    m0 = cand & (lane == j0)
                    plsc.store_scatter(match_v, [uvec], uvec, mask=m0)
                    plsc.store_scatter(match_v, [vvec], uvec, mask=m0)
                    return jnp.where(has, j0[0] + 1, jnp.int32(16))

                lax.while_loop(lambda ml: ml < 16, wbody, jnp.int32(0))
                return c2

            lax.fori_loop(0, _CH // 16, vec, 0)
            return carry

        lax.fori_loop(0, _E // _CH, chunk, 0)
        pltpu.sync_copy(match_v, match_hbm)


@jax.jit
def _sc_match(ss, ds):
    mesh = plsc.VectorSubcoreMesh(core_axis_name="c", subcore_axis_name="s")
    return pl.kernel(
        _match_body,
        mesh=mesh,
        compiler_params=pltpu.CompilerParams(needs_layout_passes=False),
        out_type=jax.ShapeDtypeStruct((_N,), jnp.int32),
        scratch_types=[
            pltpu.VMEM((_N,), jnp.int32),
            pltpu.VMEM((_CH,), jnp.int32),
            pltpu.VMEM((_CH,), jnp.int32),
        ],
    )(ss, ds)


_R = 128
_NR = _E // _R
_RPW = (_NR + _NW - 1) // _NW


def _scatter_body(gi_hbm, si_hbm, cf_hbm, y_hbm, z_hbm, out_hbm,
                  giv, siv, cfv, rows, y_sh, t_sh, sem):
    ci = lax.axis_index("c")
    si = lax.axis_index("s")
    wid = ci * 16 + si

    @pl.when(si == 0)
    def _():
        pltpu.sync_copy(y_hbm, y_sh)
        pltpu.sync_copy(z_hbm, t_sh)

    plsc.subcore_barrier()

    def row_iter(k, carry):
        r = wid + k * _NW

        @pl.when(r < _NR)
        def _():
            pltpu.sync_copy(gi_hbm.at[r], giv)
            pltpu.sync_copy(si_hbm.at[r], siv)
            pltpu.sync_copy(cf_hbm.at[r], cfv)
            pltpu.async_copy(y_sh.at[giv], rows, sem).wait()

            def scale(b, c2):
                cf16 = cfv[pl.ds(b * 16, 16)]
                for j in range(16):
                    e = b * 16 + j
                    cfj = cf16[j]
                    rows[e, pl.ds(0, 16)] = rows[e, pl.ds(0, 16)] * cfj
                    rows[e, pl.ds(16, 16)] = rows[e, pl.ds(16, 16)] * cfj
                return c2

            lax.fori_loop(0, _R // 16, scale, 0)
            pltpu.sync_copy(rows, t_sh.at[siv], add=True)

        return carry

    lax.fori_loop(0, _RPW, row_iter, 0)
    plsc.subcore_barrier()

    @pl.when(si == 0)
    def _():
        pltpu.sync_copy(t_sh, out_hbm.at[ci])


@jax.jit
def _sc_scatter_rows(gi, sidx, cf, y):
    mesh = plsc.VectorSubcoreMesh(core_axis_name="c", subcore_axis_name="s")
    z = jnp.zeros((_N, _D), jnp.float32)
    parts = pl.kernel(
        _scatter_body,
        mesh=mesh,
        compiler_params=pltpu.CompilerParams(
            needs_layout_passes=False, use_tc_tiling_on_sc=False),
        out_type=jax.ShapeDtypeStruct((2, _N, _D), jnp.float32),
        scratch_types=[
            pltpu.VMEM((_R,), jnp.int32),
            pltpu.VMEM((_R,), jnp.int32),
            pltpu.VMEM((_R,), jnp.float32),
            pltpu.VMEM((_R, _D), jnp.float32),
            pltpu.VMEM_SHARED((_N, _D), jnp.float32),
            pltpu.VMEM_SHARED((_N, _D), jnp.float32),
            pltpu.SemaphoreType.DMA,
        ],
    )(gi.reshape(_NR, _R), sidx.reshape(_NR, _R), cf.reshape(_NR, _R), y, z)
    return parts[0] + parts[1]


_RB = 1000


def _mm1_body(x_ref, w0_ref, w1_ref, xw0_ref, y_ref):
    x = x_ref[...]
    xw0_ref[...] = jnp.dot(x, w0_ref[...], preferred_element_type=jnp.float32)
    y_ref[...] = jnp.dot(x, w1_ref[...], preferred_element_type=jnp.float32)


@jax.jit
def _tc_mm1(x, w0, w1):
    m = x.shape[1]
    d = w0.shape[1]
    return pl.pallas_call(
        _mm1_body,
        grid=(_N // _RB,),
        in_specs=[
            pl.BlockSpec((_RB, m), lambda i: (i, 0)),
            pl.BlockSpec((m, d), lambda i: (0, 0)),
            pl.BlockSpec((m, d), lambda i: (0, 0)),
        ],
        out_specs=[
            pl.BlockSpec((_RB, d), lambda i: (i, 0)),
            pl.BlockSpec((_RB, d), lambda i: (i, 0)),
        ],
        out_shape=[
            jax.ShapeDtypeStruct((_N, d), jnp.float32),
            jax.ShapeDtypeStruct((_N, d), jnp.float32),
        ],
    )(x, w0, w1)


def _comb1_body(xw0_ref, t_ref, b_ref, h_ref):
    h_ref[...] = jax.nn.relu(xw0_ref[...] + t_ref[...] + b_ref[0:1, :])


@jax.jit
def _tc_comb1(xw0, t, b):
    d = xw0.shape[1]
    return pl.pallas_call(
        _comb1_body,
        grid=(_N // _RB,),
        in_specs=[
            pl.BlockSpec((_RB, d), lambda i: (i, 0)),
            pl.BlockSpec((_RB, d), lambda i: (i, 0)),
            pl.BlockSpec((8, d), lambda i: (0, 0)),
        ],
        out_specs=pl.BlockSpec((_RB, d), lambda i: (i, 0)),
        out_shape=jax.ShapeDtypeStruct((_N, d), jnp.float32),
    )(xw0, t, jnp.broadcast_to(b, (8, d)))


def _mm2_body(hp_ref, t_ref, w0_ref, w1_ref, b_ref, h2_ref):
    a = jnp.dot(hp_ref[...], w0_ref[...], preferred_element_type=jnp.float32)
    b = jnp.dot(t_ref[...], w1_ref[...], preferred_element_type=jnp.float32)
    h2_ref[...] = jax.nn.relu(a + b + b_ref[0:1, :])


@jax.jit
def _tc_mm2(hp, t, w0, w1, b):
    m = hp.shape[1]
    d = w0.shape[1]
    return pl.pallas_call(
        _mm2_body,
        grid=(_N // _RB,),
        in_specs=[
            pl.BlockSpec((_RB, m), lambda i: (i, 0)),
            pl.BlockSpec((_RB, m), lambda i: (i, 0)),
            pl.BlockSpec((m, d), lambda i: (0, 0)),
            pl.BlockSpec((m, d), lambda i: (0, 0)),
            pl.BlockSpec((8, d), lambda i: (0, 0)),
        ],
        out_specs=pl.BlockSpec((_RB, d), lambda i: (i, 0)),
        out_shape=jax.ShapeDtypeStruct((_N, d), jnp.float32),
    )(hp, t, w0, w1, jnp.broadcast_to(b, (8, d)))


def _head_body(g_ref, f1w_ref, f1b_ref, f2w_ref, f2b_ref, o_ref):
    g1 = jax.nn.relu(
        jnp.dot(g_ref[...], f1w_ref[...], preferred_element_type=jnp.float32)
        + f1b_ref[...])
    o_ref[...] = jnp.dot(
        g1, f2w_ref[...], preferred_element_type=jnp.float32) + f2b_ref[...]


@jax.jit
def _tc_head(g, f1w, f1b, f2w, f2b):
    return pl.pallas_call(
        _head_body,
        out_shape=jax.ShapeDtypeStruct((1, _N), jnp.float32),
    )(g, f1w, f1b.reshape(1, -1), f2w, f2b.reshape(1, -1))


def kernel(x, edge_index, batch, W1, b1, W2, b2, fc1_w, fc1_b, fc2_w, fc2_b):
    n = x.shape[0]
    idx = jnp.arange(n, dtype=jnp.int32)
    src = edge_index[0]
    dst = edge_index[1]
    order = jnp.argsort(src)
    ss = src[order]
    ds = dst[order]

    match1 = _sc_match(ss, ds)
    cluster1 = jnp.where(match1 == -1, idx, match1)
    head1 = cluster1 == idx
    h1i = head1.astype(jnp.int32)
    rank1 = jnp.cumsum(h1i) - h1i
    n1 = jnp.sum(h1i)
    inv1 = rank1[cluster1]
    partner1 = idx.at[jnp.where(head1, n, cluster1)].set(idx, mode="drop")

    e0 = inv1[ss]
    e1 = inv1[ds]
    valid = e0 != e1
    big = jnp.int32(n * n)
    k = jnp.where(valid, e0 * n1 + e1, big)
    ks = jnp.sort(k)
    kv = ks < big
    first = jnp.concatenate([jnp.ones((1,), bool), ks[1:] != ks[:-1]])
    keep = first & kv
    es = jnp.where(kv, ks // n1, 0).astype(jnp.int32)
    ed = jnp.where(kv, ks % n1, 0).astype(jnp.int32)

    match2 = _sc_match(es, ed)
    cluster2 = jnp.where(match2 == -1, idx, match2)
    head2 = (cluster2 == idx) & (idx < n1)
    n2 = jnp.sum(head2.astype(jnp.int32))
    partner2 = idx.at[jnp.where(head2 | (idx >= n1), n, cluster2)].set(
        idx, mode="drop")

    xf = x.astype(jnp.float32)
    pos = jnp.arange(n + 1, dtype=jnp.int32)
    row_start = jnp.searchsorted(ss, pos).astype(jnp.int32)
    deg1 = (row_start[1:] - row_start[:-1]).astype(jnp.float32)
    dis1 = jnp.where(deg1 > 0, lax.rsqrt(jnp.maximum(deg1, 1e-12)), 0.0)
    coef1 = -dis1[ss] * dis1[ds]
    xw0, y = _tc_mm1(xf, W1[0], W1[1])
    t1 = _sc_scatter_rows(ss, ds, coef1, y)
    h = _tc_comb1(xw0, t1, b1)

    hp_nodes = jnp.maximum(h, h[partner1])
    hp = jnp.zeros_like(h).at[jnp.where(head1, rank1, n)].set(
        hp_nodes, mode="drop")

    w2v = keep.astype(jnp.float32)
    deg2 = jnp.zeros((n,), jnp.float32).at[es].add(w2v)
    dis2 = jnp.where(deg2 > 0, lax.rsqrt(jnp.maximum(deg2, 1e-12)), 0.0)
    coef2 = -dis2[es] * dis2[ed] * w2v
    spread = jnp.arange(_E, dtype=jnp.int32) % jnp.int32(n)
    gi2 = jnp.where(kv, es, spread)
    si2 = jnp.where(kv, ed, spread)
    t2 = _sc_scatter_rows(gi2, si2, coef2, hp)
    h2 = _tc_mm2(hp, t2, W2[0], W2[1], b2)

    h2p = jnp.maximum(h2, h2[partner2])
    g = jnp.sum(jnp.where(head2[:, None], h2p, 0.0), axis=0,
                keepdims=True) / n2.astype(jnp.float32)

    return _tc_head(g, fc1_w, fc1_b, fc2_w, fc2_b)


# speedup vs baseline: 2411.9317x; 1.0020x over previous
import jax
import jax.numpy as jnp
from jax import lax
from jax.experimental import pallas as pl
from jax.experimental.pallas import tpu as pltpu
from jax.experimental.pallas import tpu_sc as plsc

_N = 10000
_E = 320000
_CH = 32000
_D = 32
_NW = 32


def _match_body(ss_hbm, ds_hbm, match_hbm, match_v, ssv, dsv):
    c = lax.axis_index("c")
    s = lax.axis_index("s")

    @pl.when((c == 0) & (s == 0))
    def _():
        lane = lax.iota(jnp.int32, 16)
        neg1 = jnp.full((16,), -1, jnp.int32)

        def init(i, carry):
            match_v[pl.ds(i * 16, 16)] = neg1
            return carry

        lax.fori_loop(0, _N // 16, init, 0)

        def chunk(ci, carry):
            pltpu.sync_copy(ss_hbm.at[pl.ds(ci * _CH, _CH)], ssv)
            pltpu.sync_copy(ds_hbm.at[pl.ds(ci * _CH, _CH)], dsv)

            def vec(i, c2):
                uvec = ssv[pl.ds(i * 16, 16)]
                vvec = dsv[pl.ds(i * 16, 16)]
                gt = vvec > uvec

                def wbody(minlane):
                    mu = plsc.load_gather(match_v, [uvec])
                    mv = plsc.load_gather(match_v, [vvec])
                    cand = ((mu == neg1) & gt & (mv == neg1)
                            & (lane >= minlane))
                    has = jnp.any(cand)
                    j0 = plsc.all_reduce_ffs(cand) + jnp.zeros(
                        (16,), jnp.int32)
                    m0 = cand & (lane == j0)
                    plsc.store_scatter(match_v, [uvec], uvec, mask=m0)
                    plsc.store_scatter(match_v, [vvec], uvec, mask=m0)
                    return jnp.where(has, j0[0] + 1, jnp.int32(16))

                lax.while_loop(lambda ml: ml < 16, wbody, jnp.int32(0))
                return c2

            lax.fori_loop(0, _CH // 16, vec, 0)
            return carry

        lax.fori_loop(0, _E // _CH, chunk, 0)
        pltpu.sync_copy(match_v, match_hbm)


@jax.jit
def _sc_match(ss, ds):
    mesh = plsc.VectorSubcoreMesh(core_axis_name="c", subcore_axis_name="s")
    return pl.kernel(
        _match_body,
        mesh=mesh,
        compiler_params=pltpu.CompilerParams(needs_layout_passes=False),
        out_type=jax.ShapeDtypeStruct((_N,), jnp.int32),
        scratch_types=[
            pltpu.VMEM((_N,), jnp.int32),
            pltpu.VMEM((_CH,), jnp.int32),
            pltpu.VMEM((_CH,), jnp.int32),
        ],
    )(ss, ds)


_R = 128
_NR = _E // _R
_RPW = (_NR + _NW - 1) // _NW


def _scatter_body(gi_hbm, si_hbm, cf_hbm, y_hbm, z_hbm, out_hbm,
                  giv, siv, cfv, rows, y_sh, t_sh, sem):
    ci = lax.axis_index("c")
    si = lax.axis_index("s")
    wid = ci * 16 + si

    @pl.when(si == 0)
    def _():
        pltpu.sync_copy(y_hbm, y_sh)
        pltpu.sync_copy(z_hbm, t_sh)

    plsc.subcore_barrier()

    def row_iter(k, carry):
        r = wid + k * _NW

        @pl.when(r < _NR)
        def _():
            pltpu.sync_copy(gi_hbm.at[r], giv)
            pltpu.sync_copy(si_hbm.at[r], siv)
            pltpu.sync_copy(cf_hbm.at[r], cfv)
            pltpu.async_copy(y_sh.at[giv], rows, sem).wait()

            def scale(b, c2):
                cf16 = cfv[pl.ds(b * 16, 16)]
                for j in range(16):
                    e = b * 16 + j
                    cfj = cf16[j]
                    rows[e, pl.ds(0, 16)] = rows[e, pl.ds(0, 16)] * cfj
                    rows[e, pl.ds(16, 16)] = rows[e, pl.ds(16, 16)] * cfj
                return c2

            lax.fori_loop(0, _R // 16, scale, 0)
            pltpu.sync_copy(rows, t_sh.at[siv], add=True)

        return carry

    lax.fori_loop(0, _RPW, row_iter, 0)
    plsc.subcore_barrier()

    @pl.when(si == 0)
    def _():
        pltpu.sync_copy(t_sh, out_hbm.at[ci])


@jax.jit
def _sc_scatter_rows(gi, sidx, cf, y):
    mesh = plsc.VectorSubcoreMesh(core_axis_name="c", subcore_axis_name="s")
    z = jnp.zeros((_N, _D), jnp.float32)
    parts = pl.kernel(
        _scatter_body,
        mesh=mesh,
        compiler_params=pltpu.CompilerParams(
            needs_layout_passes=False, use_tc_tiling_on_sc=False),
        out_type=jax.ShapeDtypeStruct((2, _N, _D), jnp.float32),
        scratch_types=[
            pltpu.VMEM((_R,), jnp.int32),
            pltpu.VMEM((_R,), jnp.int32),
            pltpu.VMEM((_R,), jnp.float32),
            pltpu.VMEM((_R, _D), jnp.float32),
            pltpu.VMEM_SHARED((_N, _D), jnp.float32),
            pltpu.VMEM_SHARED((_N, _D), jnp.float32),
            pltpu.SemaphoreType.DMA,
        ],
    )(gi.reshape(_NR, _R), sidx.reshape(_NR, _R), cf.reshape(_NR, _R), y, z)
    return parts[0] + parts[1]


_RB = 1000


def _mm1_body(x_ref, w0_ref, w1_ref, xw0_ref, y_ref):
    x = x_ref[...]
    xw0_ref[...] = jnp.dot(x, w0_ref[...], preferred_element_type=jnp.float32)
    y_ref[...] = jnp.dot(x, w1_ref[...], preferred_element_type=jnp.float32)


@jax.jit
def _tc_mm1(x, w0, w1):
    m = x.shape[1]
    d = w0.shape[1]
    return pl.pallas_call(
        _mm1_body,
        grid=(_N // _RB,),
        in_specs=[
            pl.BlockSpec((_RB, m), lambda i: (i, 0)),
            pl.BlockSpec((m, d), lambda i: (0, 0)),
            pl.BlockSpec((m, d), lambda i: (0, 0)),
        ],
        out_specs=[
            pl.BlockSpec((_RB, d), lambda i: (i, 0)),
            pl.BlockSpec((_RB, d), lambda i: (i, 0)),
        ],
        out_shape=[
            jax.ShapeDtypeStruct((_N, d), jnp.float32),
            jax.ShapeDtypeStruct((_N, d), jnp.float32),
        ],
    )(x, w0, w1)


def _comb1_body(xw0_ref, t_ref, b_ref, h_ref):
    h_ref[...] = jax.nn.relu(xw0_ref[...] + t_ref[...] + b_ref[0:1, :])


@jax.jit
def _tc_comb1(xw0, t, b):
    d = xw0.shape[1]
    return pl.pallas_call(
        _comb1_body,
        grid=(_N // _RB,),
        in_specs=[
            pl.BlockSpec((_RB, d), lambda i: (i, 0)),
            pl.BlockSpec((_RB, d), lambda i: (i, 0)),
            pl.BlockSpec((8, d), lambda i: (0, 0)),
        ],
        out_specs=pl.BlockSpec((_RB, d), lambda i: (i, 0)),
        out_shape=jax.ShapeDtypeStruct((_N, d), jnp.float32),
    )(xw0, t, jnp.broadcast_to(b, (8, d)))


def _mm2_body(hp_ref, t_ref, w0_ref, w1_ref, b_ref, h2_ref):
    a = jnp.dot(hp_ref[...], w0_ref[...], preferred_element_type=jnp.float32)
    b = jnp.dot(t_ref[...], w1_ref[...], preferred_element_type=jnp.float32)
    h2_ref[...] = jax.nn.relu(a + b + b_ref[0:1, :])


@jax.jit
def _tc_mm2(hp, t, w0, w1, b):
    m = hp.shape[1]
    d = w0.shape[1]
    return pl.pallas_call(
        _mm2_body,
        grid=(_N // _RB,),
        in_specs=[
            pl.BlockSpec((_RB, m), lambda i: (i, 0)),
            pl.BlockSpec((_RB, m), lambda i: (i, 0)),
            pl.BlockSpec((m, d), lambda i: (0, 0)),
            pl.BlockSpec((m, d), lambda i: (0, 0)),
            pl.BlockSpec((8, d), lambda i: (0, 0)),
        ],
        out_specs=pl.BlockSpec((_RB, d), lambda i: (i, 0)),
        out_shape=jax.ShapeDtypeStruct((_N, d), jnp.float32),
    )(hp, t, w0, w1, jnp.broadcast_to(b, (8, d)))


def _head_body(g_ref, f1w_ref, f1b_ref, f2w_ref, f2b_ref, o_ref):
    g1 = jax.nn.relu(
        jnp.dot(g_ref[...], f1w_ref[...], preferred_element_type=jnp.float32)
        + f1b_ref[...])
    o_ref[...] = jnp.dot(
        g1, f2w_ref[...], preferred_element_type=jnp.float32) + f2b_ref[...]


@jax.jit
def _tc_head(g, f1w, f1b, f2w, f2b):
    return pl.pallas_call(
        _head_body,
        out_shape=jax.ShapeDtypeStruct((1, _N), jnp.float32),
    )(g, f1w, f1b.reshape(1, -1), f2w, f2b.reshape(1, -1))


def kernel(x, edge_index, batch, W1, b1, W2, b2, fc1_w, fc1_b, fc2_w, fc2_b):
    n = x.shape[0]
    idx = jnp.arange(n, dtype=jnp.int32)
    src = edge_index[0]
    dst = edge_index[1]
    ss, ds = lax.sort_key_val(src, dst, is_stable=True)

    match1 = _sc_match(ss, ds)
    cluster1 = jnp.where(match1 == -1, idx, match1)
    head1 = cluster1 == idx
    h1i = head1.astype(jnp.int32)
    rank1 = jnp.cumsum(h1i) - h1i
    n1 = jnp.sum(h1i)
    inv1 = rank1[cluster1]
    partner1 = idx.at[jnp.where(head1, n, cluster1)].set(idx, mode="drop")

    e0 = inv1[ss]
    e1 = inv1[ds]
    valid = e0 != e1
    big = jnp.int32(n * n)
    k = jnp.where(valid, e0 * n1 + e1, big)
    ks = jnp.sort(k)
    kv = ks < big
    first = jnp.concatenate([jnp.ones((1,), bool), ks[1:] != ks[:-1]])
    keep = first & kv
    es = jnp.where(kv, ks // n1, 0).astype(jnp.int32)
    ed = jnp.where(kv, ks % n1, 0).astype(jnp.int32)

    match2 = _sc_match(es, ed)
    cluster2 = jnp.where(match2 == -1, idx, match2)
    head2 = (cluster2 == idx) & (idx < n1)
    n2 = jnp.sum(head2.astype(jnp.int32))
    partner2 = idx.at[jnp.where(head2 | (idx >= n1), n, cluster2)].set(
        idx, mode="drop")

    xf = x.astype(jnp.float32)
    pos = jnp.arange(n + 1, dtype=jnp.int32)
    row_start = jnp.searchsorted(ss, pos).astype(jnp.int32)
    deg1 = (row_start[1:] - row_start[:-1]).astype(jnp.float32)
    dis1 = jnp.where(deg1 > 0, lax.rsqrt(jnp.maximum(deg1, 1e-12)), 0.0)
    coef1 = -dis1[ss] * dis1[ds]
    xw0, y = _tc_mm1(xf, W1[0], W1[1])
    t1 = _sc_scatter_rows(ss, ds, coef1, y)
    h = _tc_comb1(xw0, t1, b1)

    hp_nodes = jnp.maximum(h, h[partner1])
    hp = jnp.zeros_like(h).at[jnp.where(head1, rank1, n)].set(
        hp_nodes, mode="drop")

    w2v = keep.astype(jnp.float32)
    deg2 = jnp.zeros((n,), jnp.float32).at[es].add(w2v)
    dis2 = jnp.where(deg2 > 0, lax.rsqrt(jnp.maximum(deg2, 1e-12)), 0.0)
    coef2 = -dis2[es] * dis2[ed] * w2v
    spread = jnp.arange(_E, dtype=jnp.int32) % jnp.int32(n)
    gi2 = jnp.where(kv, es, spread)
    si2 = jnp.where(kv, ed, spread)
    t2 = _sc_scatter_rows(gi2, si2, coef2, hp)
    h2 = _tc_mm2(hp, t2, W2[0], W2[1], b2)

    h2p = jnp.maximum(h2, h2[partner2])
    g = jnp.sum(jnp.where(head2[:, None], h2p, 0.0), axis=0,
                keepdims=True) / n2.astype(jnp.float32)

    return _tc_head(g, fc1_w, fc1_b, fc2_w, fc2_b)
